# baseline (speedup 1.0000x reference)
import functools

import jax
import jax.numpy as jnp
from jax import lax
from jax.experimental import pallas as pl
from jax.experimental.pallas import tpu as pltpu
from jax.experimental.pallas import tpu_sc as plsc

B = 1024
D = 512
S = 200
SP = 208
GEN_V = 50000
GEN_VP = 51200
INP_V = 30000
OUT_V = 100000

TN = 3200
NT = GEN_VP // TN

NC = 2
NS = 16
L = 16
NW = NC * NS
NSPLIT = 4
RS = B // NSPLIT
ROWS_PER = RS // NW
CH = 3200
NCH = GEN_VP // CH
IDS_PER = B * SP // NW


def _tc_body(xbf_ref, wg_ref, bg_ref, xf_ref, wp_ref, bp_ref, sc_ref,
             e_ref, sg_ref, pv_ref, z_ref):
    i = pl.program_id(0)
    s = jnp.dot(xbf_ref[...], wg_ref[...], preferred_element_type=jnp.float32)
    s = s + bg_ref[...]
    e = jnp.exp(s)
    col = i * TN + lax.broadcasted_iota(jnp.int32, e.shape, 1)
    e = jnp.where(col < GEN_V, e, 0.0)
    e_ref[...] = e.astype(jnp.bfloat16)

    @pl.when(i == 0)
    def _():
        z_ref[...] = jnp.zeros_like(z_ref)

    z_ref[:, 0:1] += jnp.sum(e, axis=1, keepdims=True)

    @pl.when(i == NT - 1)
    def _():
        interp = jax.nn.sigmoid(
            jnp.dot(xf_ref[...], wp_ref[...], preferred_element_type=jnp.float32)
            + bp_ref[0, 0])
        sg_ref[...] = jnp.broadcast_to(interp / z_ref[:, 0:1], (B, L))
        sc = sc_ref[...]
        m = jnp.max(sc, axis=1, keepdims=True)
        es = jnp.exp(sc - m)
        a = es / jnp.sum(es, axis=1, keepdims=True)
        pv_ref[:, 0:S] = (1.0 - interp) * a
        pv_ref[:, S:SP] = jnp.zeros((B, SP - S), jnp.float32)


def _tc_stage(x_bf, wg_bf, bg2, x, wp, bp2, scores):
    return pl.pallas_call(
        _tc_body,
        grid=(NT,),
        in_specs=[
            pl.BlockSpec((B, D), lambda i: (0, 0)),
            pl.BlockSpec((D, TN), lambda i: (0, i)),
            pl.BlockSpec((1, TN), lambda i: (0, i)),
            pl.BlockSpec((B, D), lambda i: (0, 0)),
            pl.BlockSpec((D, 1), lambda i: (0, 0)),
            pl.BlockSpec((1, 1), lambda i: (0, 0)),
            pl.BlockSpec((B, S), lambda i: (0, 0)),
        ],
        out_specs=[
            pl.BlockSpec((B, TN), lambda i: (0, i)),
            pl.BlockSpec((B, L), lambda i: (0, 0)),
            pl.BlockSpec((B, SP), lambda i: (0, 0)),
        ],
        out_shape=[
            jax.ShapeDtypeStruct((B, GEN_VP), jnp.bfloat16),
            jax.ShapeDtypeStruct((B, L), jnp.float32),
            jax.ShapeDtypeStruct((B, SP), jnp.float32),
        ],
        scratch_shapes=[pltpu.VMEM((B, 8), jnp.float32)],
    )(x_bf, wg_bf, bg2, x, wp, bp2, scores)


_MESH = plsc.VectorSubcoreMesh(core_axis_name="c", subcore_axis_name="s")
_SC_PARAMS = pltpu.CompilerParams(
    needs_layout_passes=False, use_tc_tiling_on_sc=False)


@functools.partial(
    pl.kernel,
    mesh=_MESH,
    compiler_params=_SC_PARAMS,
    out_type=jax.ShapeDtypeStruct((B * SP,), jnp.int32),
    scratch_types=[
        pltpu.VMEM((INP_V,), jnp.int32),
        pltpu.VMEM((IDS_PER,), jnp.int32),
        pltpu.VMEM((IDS_PER,), jnp.int32),
    ],
)
def _sc_remap(tbl_hbm, ctx_hbm, out_hbm, tbl_v, in_v, out_v):
    wid = lax.axis_index("s") * NC + lax.axis_index("c")
    base = wid * IDS_PER
    pltpu.sync_copy(tbl_hbm, tbl_v)
    pltpu.sync_copy(ctx_hbm.at[pl.ds(base, IDS_PER)], in_v)

    def body(i, carry):
        idx = in_v[pl.ds(i * L, L)]
        out_v[pl.ds(i * L, L)] = plsc.load_gather(tbl_v, [idx])
        return carry

    lax.fori_loop(0, IDS_PER // L, body, 0, unroll=8)
    pltpu.sync_copy(out_v, out_hbm.at[pl.ds(base, IDS_PER)])


def _make_scatter(base):
    @functools.partial(
        pl.kernel,
        mesh=_MESH,
        compiler_params=_SC_PARAMS,
        out_type=jax.ShapeDtypeStruct((RS, OUT_V), jnp.float32),
        scratch_types=[
            pltpu.VMEM((OUT_V,), jnp.float32),
            pltpu.VMEM((CH,), jnp.bfloat16),
            pltpu.VMEM((CH,), jnp.int32),
            pltpu.VMEM((CH,), jnp.bfloat16),
            pltpu.VMEM((CH,), jnp.int32),
            pltpu.VMEM((SP,), jnp.float32),
            pltpu.VMEM((SP,), jnp.int32),
            pltpu.VMEM((L,), jnp.float32),
            pltpu.VMEM_SHARED((GEN_VP,), jnp.int32),
            pltpu.SemaphoreType.DMA,
            pltpu.SemaphoreType.DMA,
            pltpu.SemaphoreType.DMA,
            pltpu.SemaphoreType.DMA,
            pltpu.SemaphoreType.DMA,
        ],
    )
    def _sc_scatter(e_hbm, g2o_hbm, ctxo_hbm, pv_hbm, sg_hbm, out_hbm,
                    row_v, val_a, idx_a, val_b, idx_b, pv_v, pi_v, sg_v,
                    g2o_s, sem_va, sem_ia, sem_vb, sem_ib, sem_out):
        cid = lax.axis_index("c")
        sid = lax.axis_index("s")
        wid = sid * NC + cid
        lanes = lax.iota(jnp.int32, L)
        last_mask = lanes < (S - (SP // L - 1) * L)

        pltpu.sync_copy(g2o_hbm.at[pl.ds(sid * (GEN_VP // NS), GEN_VP // NS)],
                        g2o_s.at[pl.ds(sid * (GEN_VP // NS), GEN_VP // NS)])
        plsc.subcore_barrier()

        def start(b, c, val_v, idx_v, sem_v, sem_i):
            pltpu.async_copy(e_hbm.at[b, pl.ds(c * CH, CH)], val_v, sem_v)
            pltpu.async_copy(g2o_s.at[pl.ds(c * CH, CH)], idx_v, sem_i)

        def wait(b, val_v, idx_v, sem_v, sem_i):
            pltpu.make_async_copy(
                e_hbm.at[b, pl.ds(0, CH)], val_v, sem_v).wait()
            pltpu.make_async_copy(
                g2o_s.at[pl.ds(0, CH)], idx_v, sem_i).wait()

        def scatter_chunk(val_v, idx_v, sg):
            def vbody(i, c2):
                ev = val_v[pl.ds(i * 2 * L, 2 * L)]
                va, vb = plsc.unpack(ev, format=plsc.PackFormat.INTERLEAVED)
                ia = idx_v[pl.ds(i * 2 * L, L)]
                ib = idx_v[pl.ds(i * 2 * L + L, L)]
                plsc.addupdate_scatter(row_v, [ia], va * sg)
                plsc.addupdate_scatter(row_v, [ib], vb * sg)
                return c2

            lax.fori_loop(0, CH // (2 * L), vbody, 0, unroll=8)

        def row_body(r, carry):
            bl = wid * ROWS_PER + r
            b = base + bl

            start(b, 0, val_a, idx_a, sem_va, sem_ia)

            pltpu.make_async_copy(row_v, out_hbm.at[bl], sem_out).wait()

            def zbody(i, c):
                row_v[pl.ds(i * L, L)] = jnp.zeros((L,), jnp.float32)
                return c

            lax.fori_loop(0, OUT_V // L, zbody, 0, unroll=8)

            pltpu.sync_copy(sg_hbm.at[b], sg_v)
            sg = sg_v[...]

            def pair_body(g, carry2):
                start(b, 2 * g + 1, val_b, idx_b, sem_vb, sem_ib)
                wait(b, val_a, idx_a, sem_va, sem_ia)
                scatter_chunk(val_a, idx_a, sg)

                @pl.when(g < NCH // 2 - 1)
                def _():
                    start(b, 2 * g + 2, val_a, idx_a, sem_va, sem_ia)

                wait(b, val_b, idx_b, sem_vb, sem_ib)
                scatter_chunk(val_b, idx_b, sg)
                return carry2

            lax.fori_loop(0, NCH // 2, pair_body, 0)

            pltpu.sync_copy(ctxo_hbm.at[pl.ds(b * SP, SP)], pi_v)
            pltpu.sync_copy(pv_hbm.at[b], pv_v)
            for j in range(SP // L):
                iv = pi_v[pl.ds(j * L, L)]
                vv = pv_v[pl.ds(j * L, L)]
                if j < SP // L - 1:
                    plsc.addupdate_scatter(row_v, [iv], vv)
                else:
                    plsc.addupdate_scatter(row_v, [iv], vv, mask=last_mask)

            pltpu.async_copy(row_v, out_hbm.at[bl], sem_out)
            return carry

        pltpu.async_copy(row_v, out_hbm.at[wid * ROWS_PER], sem_out)
        lax.fori_loop(0, ROWS_PER, row_body, 0)
        pltpu.make_async_copy(row_v, out_hbm.at[0], sem_out).wait()

    return _sc_scatter


_SCATTERS = [_make_scatter(k * RS) for k in range(NSPLIT)]


def kernel(x, scores, ctx_inp, Wp, bp, Wg, bg, gen_to_out, inp_to_out):
    x_bf = x.astype(jnp.bfloat16)
    wg_bf = Wg.astype(jnp.bfloat16)
    bg2 = bg.reshape(1, GEN_V)
    bp2 = bp.reshape(1, 1)
    ctx_pad = jnp.pad(ctx_inp, ((0, 0), (0, SP - S))).reshape(-1)
    g2o_p = jnp.pad(gen_to_out, (0, GEN_VP - GEN_V))
    g2o_r = g2o_p.reshape(-1, L, 2).transpose(0, 2, 1).reshape(-1)

    e, sg, pv = _tc_stage(x_bf, wg_bf, bg2, x, Wp, bp2, scores)
    ctx_out = _sc_remap(inp_to_out, ctx_pad)
    outs = [s(e, g2o_r, ctx_out, pv, sg) for s in _SCATTERS]
    out = jnp.zeros((B, OUT_V), jnp.float32)
    for k, piece in enumerate(outs):
        out = lax.dynamic_update_slice(out, piece, (k * RS, 0))
    return out


# speedup vs baseline: 1.0373x; 1.0373x over previous
import functools

import jax
import jax.numpy as jnp
from jax import lax
from jax.experimental import pallas as pl
from jax.experimental.pallas import tpu as pltpu
from jax.experimental.pallas import tpu_sc as plsc

B = 1024
D = 512
S = 200
SP = 208
GEN_V = 50000
GEN_VP = 51200
INP_V = 30000
OUT_V = 100000

TN = 3200
NT = GEN_VP // TN

NC = 2
NS = 16
L = 16
NW = NC * NS
NSPLIT = 4
RS = B // NSPLIT
ROWS_PER = RS // NW
CH = 3200
NCH = GEN_VP // CH
IDS_PER = B * SP // NW


def _tc_body(xbf_ref, wg_ref, bg_ref, xf_ref, wp_ref, bp_ref, sc_ref,
             e0_ref, e1_ref, e2_ref, e3_ref, sg_ref, pv_ref, z_ref):
    i = pl.program_id(0)
    s = jnp.dot(xbf_ref[...], wg_ref[...], preferred_element_type=jnp.float32)
    s = s + bg_ref[...]
    e = jnp.exp(s)
    col = i * TN + lax.broadcasted_iota(jnp.int32, e.shape, 1)
    e = jnp.where(col < GEN_V, e, 0.0)
    eb = e.astype(jnp.bfloat16)
    for k, er in enumerate((e0_ref, e1_ref, e2_ref, e3_ref)):
        er[...] = eb[k * RS:(k + 1) * RS]

    @pl.when(i == 0)
    def _():
        z_ref[...] = jnp.zeros_like(z_ref)

    z_ref[:, 0:1] += jnp.sum(e, axis=1, keepdims=True)

    @pl.when(i == NT - 1)
    def _():
        interp = jax.nn.sigmoid(
            jnp.dot(xf_ref[...], wp_ref[...], preferred_element_type=jnp.float32)
            + bp_ref[0, 0])
        sg_ref[...] = jnp.broadcast_to(interp / z_ref[:, 0:1], (B, L))
        sc = sc_ref[...]
        m = jnp.max(sc, axis=1, keepdims=True)
        es = jnp.exp(sc - m)
        a = es / jnp.sum(es, axis=1, keepdims=True)
        pv_ref[:, 0:S] = (1.0 - interp) * a
        pv_ref[:, S:SP] = jnp.zeros((B, SP - S), jnp.float32)


def _tc_stage(x_bf, wg_bf, bg2, x, wp, bp2, scores):
    return pl.pallas_call(
        _tc_body,
        grid=(NT,),
        in_specs=[
            pl.BlockSpec((B, D), lambda i: (0, 0)),
            pl.BlockSpec((D, TN), lambda i: (0, i)),
            pl.BlockSpec((1, TN), lambda i: (0, i)),
            pl.BlockSpec((B, D), lambda i: (0, 0)),
            pl.BlockSpec((D, 1), lambda i: (0, 0)),
            pl.BlockSpec((1, 1), lambda i: (0, 0)),
            pl.BlockSpec((B, S), lambda i: (0, 0)),
        ],
        out_specs=[pl.BlockSpec((RS, TN), lambda i: (0, i))] * NSPLIT + [
            pl.BlockSpec((B, L), lambda i: (0, 0)),
            pl.BlockSpec((B, SP), lambda i: (0, 0)),
        ],
        out_shape=[jax.ShapeDtypeStruct((RS, GEN_VP), jnp.bfloat16)] * NSPLIT + [
            jax.ShapeDtypeStruct((B, L), jnp.float32),
            jax.ShapeDtypeStruct((B, SP), jnp.float32),
        ],
        scratch_shapes=[pltpu.VMEM((B, 8), jnp.float32)],
    )(x_bf, wg_bf, bg2, x, wp, bp2, scores)


_MESH = plsc.VectorSubcoreMesh(core_axis_name="c", subcore_axis_name="s")
_SC_PARAMS = pltpu.CompilerParams(
    needs_layout_passes=False, use_tc_tiling_on_sc=False)


@functools.partial(
    pl.kernel,
    mesh=_MESH,
    compiler_params=_SC_PARAMS,
    out_type=jax.ShapeDtypeStruct((B * SP,), jnp.int32),
    scratch_types=[
        pltpu.VMEM((INP_V,), jnp.int32),
        pltpu.VMEM((IDS_PER,), jnp.int32),
        pltpu.VMEM((IDS_PER,), jnp.int32),
    ],
)
def _sc_remap(tbl_hbm, ctx_hbm, out_hbm, tbl_v, in_v, out_v):
    wid = lax.axis_index("s") * NC + lax.axis_index("c")
    base = wid * IDS_PER
    pltpu.sync_copy(tbl_hbm, tbl_v)
    pltpu.sync_copy(ctx_hbm.at[pl.ds(base, IDS_PER)], in_v)

    def body(i, carry):
        idx = in_v[pl.ds(i * L, L)]
        out_v[pl.ds(i * L, L)] = plsc.load_gather(tbl_v, [idx])
        return carry

    lax.fori_loop(0, IDS_PER // L, body, 0, unroll=8)
    pltpu.sync_copy(out_v, out_hbm.at[pl.ds(base, IDS_PER)])


def _make_scatter(base):
    @functools.partial(
        pl.kernel,
        mesh=_MESH,
        compiler_params=_SC_PARAMS,
        out_type=jax.ShapeDtypeStruct((RS, OUT_V), jnp.float32),
        scratch_types=[
            pltpu.VMEM((OUT_V,), jnp.float32),
            pltpu.VMEM((CH,), jnp.bfloat16),
            pltpu.VMEM((CH,), jnp.int32),
            pltpu.VMEM((CH,), jnp.bfloat16),
            pltpu.VMEM((CH,), jnp.int32),
            pltpu.VMEM((SP,), jnp.float32),
            pltpu.VMEM((SP,), jnp.int32),
            pltpu.VMEM((L,), jnp.float32),
            pltpu.VMEM_SHARED((GEN_VP,), jnp.int32),
            pltpu.SemaphoreType.DMA,
            pltpu.SemaphoreType.DMA,
            pltpu.SemaphoreType.DMA,
            pltpu.SemaphoreType.DMA,
            pltpu.SemaphoreType.DMA,
        ],
    )
    def _sc_scatter(e_hbm, g2o_hbm, ctxo_hbm, pv_hbm, sg_hbm, out_hbm,
                    row_v, val_a, idx_a, val_b, idx_b, pv_v, pi_v, sg_v,
                    g2o_s, sem_va, sem_ia, sem_vb, sem_ib, sem_out):
        cid = lax.axis_index("c")
        sid = lax.axis_index("s")
        wid = sid * NC + cid
        lanes = lax.iota(jnp.int32, L)
        last_mask = lanes < (S - (SP // L - 1) * L)

        pltpu.sync_copy(g2o_hbm.at[pl.ds(sid * (GEN_VP // NS), GEN_VP // NS)],
                        g2o_s.at[pl.ds(sid * (GEN_VP // NS), GEN_VP // NS)])
        plsc.subcore_barrier()

        def start(b, c, val_v, idx_v, sem_v, sem_i):
            pltpu.async_copy(e_hbm.at[b, pl.ds(c * CH, CH)], val_v, sem_v)
            pltpu.async_copy(g2o_s.at[pl.ds(c * CH, CH)], idx_v, sem_i)

        def wait(b, val_v, idx_v, sem_v, sem_i):
            pltpu.make_async_copy(
                e_hbm.at[b, pl.ds(0, CH)], val_v, sem_v).wait()
            pltpu.make_async_copy(
                g2o_s.at[pl.ds(0, CH)], idx_v, sem_i).wait()

        def scatter_chunk(val_v, idx_v, sg):
            def vbody(i, c2):
                ev = val_v[pl.ds(i * 2 * L, 2 * L)]
                va, vb = plsc.unpack(ev, format=plsc.PackFormat.INTERLEAVED)
                ia = idx_v[pl.ds(i * 2 * L, L)]
                ib = idx_v[pl.ds(i * 2 * L + L, L)]
                plsc.addupdate_scatter(row_v, [ia], va * sg)
                plsc.addupdate_scatter(row_v, [ib], vb * sg)
                return c2

            lax.fori_loop(0, CH // (2 * L), vbody, 0, unroll=8)

        def row_body(r, carry):
            bl = wid * ROWS_PER + r
            b = base + bl

            start(bl, 0, val_a, idx_a, sem_va, sem_ia)

            pltpu.make_async_copy(row_v, out_hbm.at[bl], sem_out).wait()

            def zbody(i, c):
                row_v[pl.ds(i * L, L)] = jnp.zeros((L,), jnp.float32)
                return c

            lax.fori_loop(0, OUT_V // L, zbody, 0, unroll=8)

            pltpu.sync_copy(sg_hbm.at[b], sg_v)
            sg = sg_v[...]

            def pair_body(g, carry2):
                start(bl, 2 * g + 1, val_b, idx_b, sem_vb, sem_ib)
                wait(bl, val_a, idx_a, sem_va, sem_ia)
                scatter_chunk(val_a, idx_a, sg)

                @pl.when(g < NCH // 2 - 1)
                def _():
                    start(bl, 2 * g + 2, val_a, idx_a, sem_va, sem_ia)

                wait(bl, val_b, idx_b, sem_vb, sem_ib)
                scatter_chunk(val_b, idx_b, sg)
                return carry2

            lax.fori_loop(0, NCH // 2, pair_body, 0)

            pltpu.sync_copy(ctxo_hbm.at[pl.ds(b * SP, SP)], pi_v)
            pltpu.sync_copy(pv_hbm.at[b], pv_v)
            for j in range(SP // L):
                iv = pi_v[pl.ds(j * L, L)]
                vv = pv_v[pl.ds(j * L, L)]
                if j < SP // L - 1:
                    plsc.addupdate_scatter(row_v, [iv], vv)
                else:
                    plsc.addupdate_scatter(row_v, [iv], vv, mask=last_mask)

            pltpu.async_copy(row_v, out_hbm.at[bl], sem_out)
            return carry

        pltpu.async_copy(row_v, out_hbm.at[wid * ROWS_PER], sem_out)
        lax.fori_loop(0, ROWS_PER, row_body, 0)
        pltpu.make_async_copy(row_v, out_hbm.at[0], sem_out).wait()

    return _sc_scatter


_SCATTERS = [_make_scatter(k * RS) for k in range(NSPLIT)]


def kernel(x, scores, ctx_inp, Wp, bp, Wg, bg, gen_to_out, inp_to_out):
    x_bf = x.astype(jnp.bfloat16)
    wg_bf = Wg.astype(jnp.bfloat16)
    bg2 = bg.reshape(1, GEN_V)
    bp2 = bp.reshape(1, 1)
    ctx_pad = jnp.pad(ctx_inp, ((0, 0), (0, SP - S))).reshape(-1)
    g2o_p = jnp.pad(gen_to_out, (0, GEN_VP - GEN_V))
    g2o_r = g2o_p.reshape(-1, L, 2).transpose(0, 2, 1).reshape(-1)

    e0, e1, e2, e3, sg, pv = _tc_stage(x_bf, wg_bf, bg2, x, Wp, bp2, scores)
    ctx_out = _sc_remap(inp_to_out, ctx_pad)
    outs = [s(ek, g2o_r, ctx_out, pv, sg)
            for s, ek in zip(_SCATTERS, (e0, e1, e2, e3))]
    return jnp.concatenate(outs, axis=0)


# speedup vs baseline: 1.0526x; 1.0148x over previous
import functools

import jax
import jax.numpy as jnp
from jax import lax
from jax.experimental import pallas as pl
from jax.experimental.pallas import tpu as pltpu
from jax.experimental.pallas import tpu_sc as plsc

B = 1024
D = 512
S = 200
SP = 208
GEN_V = 50000
GEN_VP = 51200
INP_V = 30000
OUT_V = 100000

TN = 3200
NT = GEN_VP // TN

NC = 2
NS = 16
L = 16
NW = NC * NS
NSPLIT = 8
RS = B // NSPLIT
ROWS_PER = RS // NW
CH = 3200
NCH = GEN_VP // CH
IDS_PER = B * SP // NW


def _tc_body(*refs):
    (xbf_ref, wg_ref, bg_ref, xf_ref, wp_ref, bp_ref, sc_ref) = refs[:7]
    e_refs = refs[7:7 + NSPLIT]
    sg_ref, pv_ref, z_ref = refs[7 + NSPLIT:]
    i = pl.program_id(0)
    s = jnp.dot(xbf_ref[...], wg_ref[...], preferred_element_type=jnp.float32)
    s = s + bg_ref[...]
    e = jnp.exp(s)
    col = i * TN + lax.broadcasted_iota(jnp.int32, e.shape, 1)
    e = jnp.where(col < GEN_V, e, 0.0)
    eb = e.astype(jnp.bfloat16)
    for k, er in enumerate(e_refs):
        er[...] = eb[k * RS:(k + 1) * RS]

    @pl.when(i == 0)
    def _():
        z_ref[...] = jnp.zeros_like(z_ref)

    z_ref[:, 0:1] += jnp.sum(e, axis=1, keepdims=True)

    @pl.when(i == NT - 1)
    def _():
        interp = jax.nn.sigmoid(
            jnp.dot(xf_ref[...], wp_ref[...], preferred_element_type=jnp.float32)
            + bp_ref[0, 0])
        sg_ref[...] = jnp.broadcast_to(interp / z_ref[:, 0:1], (B, L))
        sc = sc_ref[...]
        m = jnp.max(sc, axis=1, keepdims=True)
        es = jnp.exp(sc - m)
        a = es / jnp.sum(es, axis=1, keepdims=True)
        pv_ref[:, 0:S] = (1.0 - interp) * a
        pv_ref[:, S:SP] = jnp.zeros((B, SP - S), jnp.float32)


def _tc_stage(x_bf, wg_bf, bg2, x, wp, bp2, scores):
    return pl.pallas_call(
        _tc_body,
        grid=(NT,),
        in_specs=[
            pl.BlockSpec((B, D), lambda i: (0, 0)),
            pl.BlockSpec((D, TN), lambda i: (0, i)),
            pl.BlockSpec((1, TN), lambda i: (0, i)),
            pl.BlockSpec((B, D), lambda i: (0, 0)),
            pl.BlockSpec((D, 1), lambda i: (0, 0)),
            pl.BlockSpec((1, 1), lambda i: (0, 0)),
            pl.BlockSpec((B, S), lambda i: (0, 0)),
        ],
        out_specs=[pl.BlockSpec((RS, TN), lambda i: (0, i))] * NSPLIT + [
            pl.BlockSpec((B, L), lambda i: (0, 0)),
            pl.BlockSpec((B, SP), lambda i: (0, 0)),
        ],
        out_shape=[jax.ShapeDtypeStruct((RS, GEN_VP), jnp.bfloat16)] * NSPLIT + [
            jax.ShapeDtypeStruct((B, L), jnp.float32),
            jax.ShapeDtypeStruct((B, SP), jnp.float32),
        ],
        scratch_shapes=[pltpu.VMEM((B, 8), jnp.float32)],
    )(x_bf, wg_bf, bg2, x, wp, bp2, scores)


_MESH = plsc.VectorSubcoreMesh(core_axis_name="c", subcore_axis_name="s")
_SC_PARAMS = pltpu.CompilerParams(
    needs_layout_passes=False, use_tc_tiling_on_sc=False)


@functools.partial(
    pl.kernel,
    mesh=_MESH,
    compiler_params=_SC_PARAMS,
    out_type=jax.ShapeDtypeStruct((B * SP,), jnp.int32),
    scratch_types=[
        pltpu.VMEM((INP_V,), jnp.int32),
        pltpu.VMEM((IDS_PER,), jnp.int32),
        pltpu.VMEM((IDS_PER,), jnp.int32),
    ],
)
def _sc_remap(tbl_hbm, ctx_hbm, out_hbm, tbl_v, in_v, out_v):
    wid = lax.axis_index("s") * NC + lax.axis_index("c")
    base = wid * IDS_PER
    pltpu.sync_copy(tbl_hbm, tbl_v)
    pltpu.sync_copy(ctx_hbm.at[pl.ds(base, IDS_PER)], in_v)

    def body(i, carry):
        idx = in_v[pl.ds(i * L, L)]
        out_v[pl.ds(i * L, L)] = plsc.load_gather(tbl_v, [idx])
        return carry

    lax.fori_loop(0, IDS_PER // L, body, 0, unroll=8)
    pltpu.sync_copy(out_v, out_hbm.at[pl.ds(base, IDS_PER)])


def _make_scatter(base):
    @functools.partial(
        pl.kernel,
        mesh=_MESH,
        compiler_params=_SC_PARAMS,
        out_type=jax.ShapeDtypeStruct((RS, OUT_V), jnp.float32),
        scratch_types=[
            pltpu.VMEM((OUT_V,), jnp.float32),
            pltpu.VMEM((CH,), jnp.bfloat16),
            pltpu.VMEM((CH,), jnp.int32),
            pltpu.VMEM((CH,), jnp.bfloat16),
            pltpu.VMEM((CH,), jnp.int32),
            pltpu.VMEM((SP,), jnp.float32),
            pltpu.VMEM((SP,), jnp.int32),
            pltpu.VMEM((L,), jnp.float32),
            pltpu.VMEM_SHARED((GEN_VP,), jnp.int32),
            pltpu.SemaphoreType.DMA,
            pltpu.SemaphoreType.DMA,
            pltpu.SemaphoreType.DMA,
            pltpu.SemaphoreType.DMA,
            pltpu.SemaphoreType.DMA,
        ],
    )
    def _sc_scatter(e_hbm, g2o_hbm, ctxo_hbm, pv_hbm, sg_hbm, out_hbm,
                    row_v, val_a, idx_a, val_b, idx_b, pv_v, pi_v, sg_v,
                    g2o_s, sem_va, sem_ia, sem_vb, sem_ib, sem_out):
        cid = lax.axis_index("c")
        sid = lax.axis_index("s")
        wid = sid * NC + cid
        lanes = lax.iota(jnp.int32, L)
        last_mask = lanes < (S - (SP // L - 1) * L)

        pltpu.sync_copy(g2o_hbm.at[pl.ds(sid * (GEN_VP // NS), GEN_VP // NS)],
                        g2o_s.at[pl.ds(sid * (GEN_VP // NS), GEN_VP // NS)])
        plsc.subcore_barrier()

        def start(b, c, val_v, idx_v, sem_v, sem_i):
            pltpu.async_copy(e_hbm.at[b, pl.ds(c * CH, CH)], val_v, sem_v)
            pltpu.async_copy(g2o_s.at[pl.ds(c * CH, CH)], idx_v, sem_i)

        def wait(b, val_v, idx_v, sem_v, sem_i):
            pltpu.make_async_copy(
                e_hbm.at[b, pl.ds(0, CH)], val_v, sem_v).wait()
            pltpu.make_async_copy(
                g2o_s.at[pl.ds(0, CH)], idx_v, sem_i).wait()

        def scatter_chunk(val_v, idx_v, sg):
            def vbody(i, c2):
                ev = val_v[pl.ds(i * 2 * L, 2 * L)]
                va, vb = plsc.unpack(ev, format=plsc.PackFormat.INTERLEAVED)
                ia = idx_v[pl.ds(i * 2 * L, L)]
                ib = idx_v[pl.ds(i * 2 * L + L, L)]
                plsc.addupdate_scatter(row_v, [ia], va * sg)
                plsc.addupdate_scatter(row_v, [ib], vb * sg)
                return c2

            lax.fori_loop(0, CH // (2 * L), vbody, 0, unroll=8)

        def row_body(r, carry):
            bl = wid * ROWS_PER + r
            b = base + bl

            start(bl, 0, val_a, idx_a, sem_va, sem_ia)

            pltpu.make_async_copy(row_v, out_hbm.at[bl], sem_out).wait()

            def zbody(i, c):
                row_v[pl.ds(i * L, L)] = jnp.zeros((L,), jnp.float32)
                return c

            lax.fori_loop(0, OUT_V // L, zbody, 0, unroll=8)

            pltpu.sync_copy(sg_hbm.at[b], sg_v)
            sg = sg_v[...]

            def pair_body(g, carry2):
                start(bl, 2 * g + 1, val_b, idx_b, sem_vb, sem_ib)
                wait(bl, val_a, idx_a, sem_va, sem_ia)
                scatter_chunk(val_a, idx_a, sg)

                @pl.when(g < NCH // 2 - 1)
                def _():
                    start(bl, 2 * g + 2, val_a, idx_a, sem_va, sem_ia)

                wait(bl, val_b, idx_b, sem_vb, sem_ib)
                scatter_chunk(val_b, idx_b, sg)
                return carry2

            lax.fori_loop(0, NCH // 2, pair_body, 0)

            pltpu.sync_copy(ctxo_hbm.at[pl.ds(b * SP, SP)], pi_v)
            pltpu.sync_copy(pv_hbm.at[b], pv_v)
            for j in range(SP // L):
                iv = pi_v[pl.ds(j * L, L)]
                vv = pv_v[pl.ds(j * L, L)]
                if j < SP // L - 1:
                    plsc.addupdate_scatter(row_v, [iv], vv)
                else:
                    plsc.addupdate_scatter(row_v, [iv], vv, mask=last_mask)

            pltpu.async_copy(row_v, out_hbm.at[bl], sem_out)
            return carry

        pltpu.async_copy(row_v, out_hbm.at[wid * ROWS_PER], sem_out)
        lax.fori_loop(0, ROWS_PER, row_body, 0)
        pltpu.make_async_copy(row_v, out_hbm.at[0], sem_out).wait()

    return _sc_scatter


_SCATTERS = [_make_scatter(k * RS) for k in range(NSPLIT)]


def kernel(x, scores, ctx_inp, Wp, bp, Wg, bg, gen_to_out, inp_to_out):
    x_bf = x.astype(jnp.bfloat16)
    wg_bf = Wg.astype(jnp.bfloat16)
    bg2 = bg.reshape(1, GEN_V)
    bp2 = bp.reshape(1, 1)
    ctx_pad = jnp.pad(ctx_inp, ((0, 0), (0, SP - S))).reshape(-1)
    g2o_p = jnp.pad(gen_to_out, (0, GEN_VP - GEN_V))
    g2o_r = g2o_p.reshape(-1, L, 2).transpose(0, 2, 1).reshape(-1)

    *e_pieces, sg, pv = _tc_stage(x_bf, wg_bf, bg2, x, Wp, bp2, scores)
    ctx_out = _sc_remap(inp_to_out, ctx_pad)
    outs = [s(ek, g2o_r, ctx_out, pv, sg)
            for s, ek in zip(_SCATTERS, e_pieces)]
    return jnp.concatenate(outs, axis=0)


# speedup vs baseline: 1.1161x; 1.0603x over previous
import functools

import jax
import jax.numpy as jnp
from jax import lax
from jax.experimental import pallas as pl
from jax.experimental.pallas import tpu as pltpu
from jax.experimental.pallas import tpu_sc as plsc

B = 1024
D = 512
S = 200
SP = 208
GEN_V = 50000
GEN_VP = 51200
INP_V = 30000
OUT_V = 100000

TN = 3200
NT = GEN_VP // TN

NC = 2
NS = 16
L = 16
NW = NC * NS
NSPLIT = 8
RS = B // NSPLIT
ROWS_PER = RS // NW
CH = 3200
NCH = GEN_VP // CH
IDS_PER = B * SP // NW


def _tc_body(*refs):
    (xbf_ref, wg_ref, bg_ref, xf_ref, wp_ref, bp_ref, sc_ref) = refs[:7]
    e_refs = refs[7:7 + NSPLIT]
    sg_ref, pv_ref, z_ref = refs[7 + NSPLIT:]
    i = pl.program_id(0)
    s = jnp.dot(xbf_ref[...], wg_ref[...], preferred_element_type=jnp.float32)
    s = s + bg_ref[...]
    e = jnp.exp(s)
    col = i * TN + lax.broadcasted_iota(jnp.int32, e.shape, 1)
    e = jnp.where(col < GEN_V, e, 0.0)
    eb = e.astype(jnp.bfloat16)
    for k, er in enumerate(e_refs):
        er[...] = eb[k * RS:(k + 1) * RS]

    @pl.when(i == 0)
    def _():
        z_ref[...] = jnp.zeros_like(z_ref)

    z_ref[:, 0:1] += jnp.sum(e, axis=1, keepdims=True)

    @pl.when(i == NT - 1)
    def _():
        interp = jax.nn.sigmoid(
            jnp.dot(xf_ref[...], wp_ref[...], preferred_element_type=jnp.float32)
            + bp_ref[0, 0])
        sg_ref[...] = jnp.broadcast_to(interp / z_ref[:, 0:1], (B, L))
        sc = sc_ref[...]
        m = jnp.max(sc, axis=1, keepdims=True)
        es = jnp.exp(sc - m)
        a = es / jnp.sum(es, axis=1, keepdims=True)
        pv_ref[:, 0:S] = (1.0 - interp) * a
        pv_ref[:, S:SP] = jnp.zeros((B, SP - S), jnp.float32)


def _tc_stage(x_bf, wg_bf, bg2, x, wp, bp2, scores):
    return pl.pallas_call(
        _tc_body,
        grid=(NT,),
        in_specs=[
            pl.BlockSpec((B, D), lambda i: (0, 0)),
            pl.BlockSpec((D, TN), lambda i: (0, i)),
            pl.BlockSpec((1, TN), lambda i: (0, i)),
            pl.BlockSpec((B, D), lambda i: (0, 0)),
            pl.BlockSpec((D, 1), lambda i: (0, 0)),
            pl.BlockSpec((1, 1), lambda i: (0, 0)),
            pl.BlockSpec((B, S), lambda i: (0, 0)),
        ],
        out_specs=[pl.BlockSpec((RS, TN), lambda i: (0, i))] * NSPLIT + [
            pl.BlockSpec((B, L), lambda i: (0, 0)),
            pl.BlockSpec((B, SP), lambda i: (0, 0)),
        ],
        out_shape=[jax.ShapeDtypeStruct((RS, GEN_VP), jnp.bfloat16)] * NSPLIT + [
            jax.ShapeDtypeStruct((B, L), jnp.float32),
            jax.ShapeDtypeStruct((B, SP), jnp.float32),
        ],
        scratch_shapes=[pltpu.VMEM((B, 8), jnp.float32)],
    )(x_bf, wg_bf, bg2, x, wp, bp2, scores)


_MESH = plsc.VectorSubcoreMesh(core_axis_name="c", subcore_axis_name="s")
_SC_PARAMS = pltpu.CompilerParams(
    needs_layout_passes=False, use_tc_tiling_on_sc=False)


@functools.partial(
    pl.kernel,
    mesh=_MESH,
    compiler_params=_SC_PARAMS,
    out_type=jax.ShapeDtypeStruct((B * SP,), jnp.int32),
    scratch_types=[
        pltpu.VMEM((INP_V,), jnp.int32),
        pltpu.VMEM((IDS_PER,), jnp.int32),
        pltpu.VMEM((IDS_PER,), jnp.int32),
    ],
)
def _sc_remap(tbl_hbm, ctx_hbm, out_hbm, tbl_v, in_v, out_v):
    wid = lax.axis_index("s") * NC + lax.axis_index("c")
    base = wid * IDS_PER
    pltpu.sync_copy(tbl_hbm, tbl_v)
    pltpu.sync_copy(ctx_hbm.at[pl.ds(base, IDS_PER)], in_v)

    def body(i, carry):
        idx = in_v[pl.ds(i * L, L)]
        out_v[pl.ds(i * L, L)] = plsc.load_gather(tbl_v, [idx])
        return carry

    lax.fori_loop(0, IDS_PER // L, body, 0, unroll=8)
    pltpu.sync_copy(out_v, out_hbm.at[pl.ds(base, IDS_PER)])


def _make_scatter(base):
    @functools.partial(
        pl.kernel,
        mesh=_MESH,
        compiler_params=_SC_PARAMS,
        out_type=jax.ShapeDtypeStruct((RS, OUT_V), jnp.float32),
        scratch_types=[
            pltpu.VMEM((OUT_V,), jnp.float32),
            pltpu.VMEM((CH,), jnp.bfloat16),
            pltpu.VMEM((CH,), jnp.int32),
            pltpu.VMEM((CH,), jnp.bfloat16),
            pltpu.VMEM((CH,), jnp.int32),
            pltpu.VMEM((SP,), jnp.float32),
            pltpu.VMEM((SP,), jnp.int32),
            pltpu.VMEM((L,), jnp.float32),
            pltpu.VMEM_SHARED((GEN_VP,), jnp.int32),
            pltpu.SemaphoreType.DMA,
            pltpu.SemaphoreType.DMA,
            pltpu.SemaphoreType.DMA,
            pltpu.SemaphoreType.DMA,
            pltpu.SemaphoreType.DMA,
        ],
    )
    def _sc_scatter(e_hbm, g2o_hbm, ctxo_hbm, pv_hbm, sg_hbm, out_hbm,
                    row_v, val_a, idx_a, val_b, idx_b, pv_v, pi_v, sg_v,
                    g2o_s, sem_va, sem_ia, sem_vb, sem_ib, sem_out):
        cid = lax.axis_index("c")
        sid = lax.axis_index("s")
        wid = sid * NC + cid
        lanes = lax.iota(jnp.int32, L)
        last_mask = lanes < (S - (SP // L - 1) * L)

        pltpu.sync_copy(g2o_hbm.at[pl.ds(sid * (GEN_VP // NS), GEN_VP // NS)],
                        g2o_s.at[pl.ds(sid * (GEN_VP // NS), GEN_VP // NS)])
        plsc.subcore_barrier()

        def start(b, c, val_v, idx_v, sem_v, sem_i):
            pltpu.async_copy(e_hbm.at[b, pl.ds(c * CH, CH)], val_v, sem_v)
            pltpu.async_copy(g2o_s.at[pl.ds(c * CH, CH)], idx_v, sem_i)

        def wait(b, val_v, idx_v, sem_v, sem_i):
            pltpu.make_async_copy(
                e_hbm.at[b, pl.ds(0, CH)], val_v, sem_v).wait()
            pltpu.make_async_copy(
                g2o_s.at[pl.ds(0, CH)], idx_v, sem_i).wait()

        def scatter_chunk(val_v, idx_v, sg):
            @plsc.parallel_loop(0, CH // (2 * L), unroll=8)
            def vbody(i):
                ev = val_v[pl.ds(i * 2 * L, 2 * L)]
                va, vb = plsc.unpack(ev, format=plsc.PackFormat.INTERLEAVED)
                ia = idx_v[pl.ds(i * 2 * L, L)]
                ib = idx_v[pl.ds(i * 2 * L + L, L)]
                plsc.addupdate_scatter(row_v, [ia], va * sg)
                plsc.addupdate_scatter(row_v, [ib], vb * sg)

        def row_body(r, carry):
            bl = wid * ROWS_PER + r
            b = base + bl

            start(bl, 0, val_a, idx_a, sem_va, sem_ia)

            pltpu.make_async_copy(row_v, out_hbm.at[bl], sem_out).wait()

            @plsc.parallel_loop(0, OUT_V // L, unroll=16)
            def zbody(i):
                row_v[pl.ds(i * L, L)] = jnp.zeros((L,), jnp.float32)

            pltpu.sync_copy(sg_hbm.at[b], sg_v)
            sg = sg_v[...]

            def pair_body(g, carry2):
                start(bl, 2 * g + 1, val_b, idx_b, sem_vb, sem_ib)
                wait(bl, val_a, idx_a, sem_va, sem_ia)
                scatter_chunk(val_a, idx_a, sg)

                @pl.when(g < NCH // 2 - 1)
                def _():
                    start(bl, 2 * g + 2, val_a, idx_a, sem_va, sem_ia)

                wait(bl, val_b, idx_b, sem_vb, sem_ib)
                scatter_chunk(val_b, idx_b, sg)
                return carry2

            lax.fori_loop(0, NCH // 2, pair_body, 0)

            pltpu.sync_copy(ctxo_hbm.at[pl.ds(b * SP, SP)], pi_v)
            pltpu.sync_copy(pv_hbm.at[b], pv_v)
            for j in range(SP // L):
                iv = pi_v[pl.ds(j * L, L)]
                vv = pv_v[pl.ds(j * L, L)]
                if j < SP // L - 1:
                    plsc.addupdate_scatter(row_v, [iv], vv)
                else:
                    plsc.addupdate_scatter(row_v, [iv], vv, mask=last_mask)

            pltpu.async_copy(row_v, out_hbm.at[bl], sem_out)
            return carry

        pltpu.async_copy(row_v, out_hbm.at[wid * ROWS_PER], sem_out)
        lax.fori_loop(0, ROWS_PER, row_body, 0)
        pltpu.make_async_copy(row_v, out_hbm.at[0], sem_out).wait()

    return _sc_scatter


_SCATTERS = [_make_scatter(k * RS) for k in range(NSPLIT)]


def kernel(x, scores, ctx_inp, Wp, bp, Wg, bg, gen_to_out, inp_to_out):
    x_bf = x.astype(jnp.bfloat16)
    wg_bf = Wg.astype(jnp.bfloat16)
    bg2 = bg.reshape(1, GEN_V)
    bp2 = bp.reshape(1, 1)
    ctx_pad = jnp.pad(ctx_inp, ((0, 0), (0, SP - S))).reshape(-1)
    g2o_p = jnp.pad(gen_to_out, (0, GEN_VP - GEN_V))
    g2o_r = g2o_p.reshape(-1, L, 2).transpose(0, 2, 1).reshape(-1)

    *e_pieces, sg, pv = _tc_stage(x_bf, wg_bf, bg2, x, Wp, bp2, scores)
    ctx_out = _sc_remap(inp_to_out, ctx_pad)
    outs = [s(ek, g2o_r, ctx_out, pv, sg)
            for s, ek in zip(_SCATTERS, e_pieces)]
    return jnp.concatenate(outs, axis=0)


# speedup vs baseline: 1.1433x; 1.0244x over previous
import functools

import jax
import jax.numpy as jnp
from jax import lax
from jax.experimental import pallas as pl
from jax.experimental.pallas import tpu as pltpu
from jax.experimental.pallas import tpu_sc as plsc

B = 1024
D = 512
S = 200
SP = 208
GEN_V = 50000
GEN_VP = 51200
INP_V = 30000
OUT_V = 100000

TN = 3200
NT = GEN_VP // TN

NC = 2
NS = 16
L = 16
NW = NC * NS
NSPLIT = 8
RS = B // NSPLIT
ROWS_PER = RS // NW
CH = 3200
NCH = GEN_VP // CH
IDS_PER = B * SP // NW


def _tc_body(*refs):
    (xbf_ref, wg_ref, bg_ref, xf_ref, wp_ref, bp_ref, sc_ref) = refs[:7]
    e_refs = refs[7:7 + NSPLIT]
    sg_ref, pv_ref, z_ref = refs[7 + NSPLIT:]
    i = pl.program_id(0)
    s = jnp.dot(xbf_ref[...], wg_ref[...], preferred_element_type=jnp.float32)
    s = s + bg_ref[...]
    e = jnp.exp(s)
    col = i * TN + lax.broadcasted_iota(jnp.int32, e.shape, 1)
    e = jnp.where(col < GEN_V, e, 0.0)
    eb = e.astype(jnp.bfloat16)
    for k, er in enumerate(e_refs):
        er[...] = eb[k * RS:(k + 1) * RS]

    @pl.when(i == 0)
    def _():
        z_ref[...] = jnp.zeros_like(z_ref)

    z_ref[:, 0:1] += jnp.sum(e, axis=1, keepdims=True)

    @pl.when(i == NT - 1)
    def _():
        interp = jax.nn.sigmoid(
            jnp.dot(xf_ref[...], wp_ref[...], preferred_element_type=jnp.float32)
            + bp_ref[0, 0])
        sg_ref[...] = jnp.broadcast_to(interp / z_ref[:, 0:1], (B, L))
        sc = sc_ref[...]
        m = jnp.max(sc, axis=1, keepdims=True)
        es = jnp.exp(sc - m)
        a = es / jnp.sum(es, axis=1, keepdims=True)
        pv_ref[:, 0:S] = (1.0 - interp) * a
        pv_ref[:, S:SP] = jnp.zeros((B, SP - S), jnp.float32)


def _tc_stage(x_bf, wg_bf, bg2, x, wp, bp2, scores):
    return pl.pallas_call(
        _tc_body,
        grid=(NT,),
        in_specs=[
            pl.BlockSpec((B, D), lambda i: (0, 0)),
            pl.BlockSpec((D, TN), lambda i: (0, i)),
            pl.BlockSpec((1, TN), lambda i: (0, i)),
            pl.BlockSpec((B, D), lambda i: (0, 0)),
            pl.BlockSpec((D, 1), lambda i: (0, 0)),
            pl.BlockSpec((1, 1), lambda i: (0, 0)),
            pl.BlockSpec((B, S), lambda i: (0, 0)),
        ],
        out_specs=[pl.BlockSpec((RS, TN), lambda i: (0, i))] * NSPLIT + [
            pl.BlockSpec((B, L), lambda i: (0, 0)),
            pl.BlockSpec((B, SP), lambda i: (0, 0)),
        ],
        out_shape=[jax.ShapeDtypeStruct((RS, GEN_VP), jnp.bfloat16)] * NSPLIT + [
            jax.ShapeDtypeStruct((B, L), jnp.float32),
            jax.ShapeDtypeStruct((B, SP), jnp.float32),
        ],
        scratch_shapes=[pltpu.VMEM((B, 8), jnp.float32)],
    )(x_bf, wg_bf, bg2, x, wp, bp2, scores)


_MESH = plsc.VectorSubcoreMesh(core_axis_name="c", subcore_axis_name="s")
_SC_PARAMS = pltpu.CompilerParams(
    needs_layout_passes=False, use_tc_tiling_on_sc=False)


@functools.partial(
    pl.kernel,
    mesh=_MESH,
    compiler_params=_SC_PARAMS,
    out_type=jax.ShapeDtypeStruct((B * SP,), jnp.int32),
    scratch_types=[
        pltpu.VMEM((INP_V,), jnp.int32),
        pltpu.VMEM((IDS_PER,), jnp.int32),
        pltpu.VMEM((IDS_PER,), jnp.int32),
    ],
)
def _sc_remap(tbl_hbm, ctx_hbm, out_hbm, tbl_v, in_v, out_v):
    wid = lax.axis_index("s") * NC + lax.axis_index("c")
    base = wid * IDS_PER
    pltpu.sync_copy(tbl_hbm, tbl_v)
    pltpu.sync_copy(ctx_hbm.at[pl.ds(base, IDS_PER)], in_v)

    def body(i, carry):
        idx = in_v[pl.ds(i * L, L)]
        out_v[pl.ds(i * L, L)] = plsc.load_gather(tbl_v, [idx])
        return carry

    lax.fori_loop(0, IDS_PER // L, body, 0, unroll=8)
    pltpu.sync_copy(out_v, out_hbm.at[pl.ds(base, IDS_PER)])


def _make_scatter(base):
    @functools.partial(
        pl.kernel,
        mesh=_MESH,
        compiler_params=_SC_PARAMS,
        out_type=jax.ShapeDtypeStruct((RS, OUT_V), jnp.float32),
        scratch_types=[
            pltpu.VMEM((OUT_V,), jnp.float32),
            pltpu.VMEM((CH,), jnp.bfloat16),
            pltpu.VMEM((CH,), jnp.int32),
            pltpu.VMEM((CH,), jnp.bfloat16),
            pltpu.VMEM((CH,), jnp.int32),
            pltpu.VMEM((ROWS_PER, SP), jnp.float32),
            pltpu.VMEM((ROWS_PER * SP,), jnp.int32),
            pltpu.VMEM((ROWS_PER, L), jnp.float32),
            pltpu.VMEM_SHARED((GEN_VP,), jnp.int32),
            pltpu.SemaphoreType.DMA,
            pltpu.SemaphoreType.DMA,
            pltpu.SemaphoreType.DMA,
            pltpu.SemaphoreType.DMA,
            pltpu.SemaphoreType.DMA,
        ],
    )
    def _sc_scatter(e_hbm, g2o_hbm, ctxo_hbm, pv_hbm, sg_hbm, out_hbm,
                    row_v, val_a, idx_a, val_b, idx_b, pv_v, pi_v, sg_v,
                    g2o_s, sem_va, sem_ia, sem_vb, sem_ib, sem_out):
        cid = lax.axis_index("c")
        sid = lax.axis_index("s")
        wid = sid * NC + cid
        lanes = lax.iota(jnp.int32, L)
        last_mask = lanes < (S - (SP // L - 1) * L)

        pltpu.sync_copy(g2o_hbm.at[pl.ds(sid * (GEN_VP // NS), GEN_VP // NS)],
                        g2o_s.at[pl.ds(sid * (GEN_VP // NS), GEN_VP // NS)])

        gbase = base + wid * ROWS_PER
        pltpu.sync_copy(sg_hbm.at[pl.ds(gbase, ROWS_PER)], sg_v)
        pltpu.sync_copy(pv_hbm.at[pl.ds(gbase, ROWS_PER)], pv_v)
        pltpu.sync_copy(ctxo_hbm.at[pl.ds(gbase * SP, ROWS_PER * SP)], pi_v)
        plsc.subcore_barrier()

        def start(b, c, val_v, idx_v, sem_v, sem_i):
            pltpu.async_copy(e_hbm.at[b, pl.ds(c * CH, CH)], val_v, sem_v)
            pltpu.async_copy(g2o_s.at[pl.ds(c * CH, CH)], idx_v, sem_i)

        def wait(b, val_v, idx_v, sem_v, sem_i):
            pltpu.make_async_copy(
                e_hbm.at[b, pl.ds(0, CH)], val_v, sem_v).wait()
            pltpu.make_async_copy(
                g2o_s.at[pl.ds(0, CH)], idx_v, sem_i).wait()

        def scatter_chunk(val_v, idx_v, sg):
            @plsc.parallel_loop(0, CH // (2 * L), unroll=8)
            def vbody(i):
                ev = val_v[pl.ds(i * 2 * L, 2 * L)]
                va, vb = plsc.unpack(ev, format=plsc.PackFormat.INTERLEAVED)
                ia = idx_v[pl.ds(i * 2 * L, L)]
                ib = idx_v[pl.ds(i * 2 * L + L, L)]
                plsc.addupdate_scatter(row_v, [ia], va * sg)
                plsc.addupdate_scatter(row_v, [ib], vb * sg)

        def row_body(r, carry):
            bl = wid * ROWS_PER + r
            b = base + bl

            start(bl, 0, val_a, idx_a, sem_va, sem_ia)

            pltpu.make_async_copy(row_v, out_hbm.at[bl], sem_out).wait()

            @plsc.parallel_loop(0, OUT_V // L, unroll=16)
            def zbody(i):
                row_v[pl.ds(i * L, L)] = jnp.zeros((L,), jnp.float32)

            sg = sg_v[r, :]

            def pair_body(g, carry2):
                start(bl, 2 * g + 1, val_b, idx_b, sem_vb, sem_ib)
                wait(bl, val_a, idx_a, sem_va, sem_ia)
                scatter_chunk(val_a, idx_a, sg)

                @pl.when(g < NCH // 2 - 1)
                def _():
                    start(bl, 2 * g + 2, val_a, idx_a, sem_va, sem_ia)

                wait(bl, val_b, idx_b, sem_vb, sem_ib)
                scatter_chunk(val_b, idx_b, sg)
                return carry2

            lax.fori_loop(0, NCH // 2, pair_body, 0)

            for j in range(SP // L):
                iv = pi_v[pl.ds(r * SP + j * L, L)]
                vv = pv_v[r, pl.ds(j * L, L)]
                if j < SP // L - 1:
                    plsc.addupdate_scatter(row_v, [iv], vv)
                else:
                    plsc.addupdate_scatter(row_v, [iv], vv, mask=last_mask)

            pltpu.async_copy(row_v, out_hbm.at[bl], sem_out)
            return carry

        pltpu.async_copy(row_v, out_hbm.at[wid * ROWS_PER], sem_out)
        lax.fori_loop(0, ROWS_PER, row_body, 0)
        pltpu.make_async_copy(row_v, out_hbm.at[0], sem_out).wait()

    return _sc_scatter


_SCATTERS = [_make_scatter(k * RS) for k in range(NSPLIT)]


def kernel(x, scores, ctx_inp, Wp, bp, Wg, bg, gen_to_out, inp_to_out):
    x_bf = x.astype(jnp.bfloat16)
    wg_bf = Wg.astype(jnp.bfloat16)
    bg2 = bg.reshape(1, GEN_V)
    bp2 = bp.reshape(1, 1)
    ctx_pad = jnp.pad(ctx_inp, ((0, 0), (0, SP - S))).reshape(-1)
    g2o_p = jnp.pad(gen_to_out, (0, GEN_VP - GEN_V))
    g2o_r = g2o_p.reshape(-1, L, 2).transpose(0, 2, 1).reshape(-1)

    *e_pieces, sg, pv = _tc_stage(x_bf, wg_bf, bg2, x, Wp, bp2, scores)
    ctx_out = _sc_remap(inp_to_out, ctx_pad)
    outs = [s(ek, g2o_r, ctx_out, pv, sg)
            for s, ek in zip(_SCATTERS, e_pieces)]
    return jnp.concatenate(outs, axis=0)


# speedup vs baseline: 1.1600x; 1.0146x over previous
import functools

import jax
import jax.numpy as jnp
from jax import lax
from jax.experimental import pallas as pl
from jax.experimental.pallas import tpu as pltpu
from jax.experimental.pallas import tpu_sc as plsc

B = 1024
D = 512
S = 200
SP = 208
GEN_V = 50000
GEN_VP = 51200
INP_V = 30000
OUT_V = 100000

TN = 3200
NT = GEN_VP // TN

NC = 2
NS = 16
L = 16
NW = NC * NS
NSPLIT = 8
RS = B // NSPLIT
ROWS_PER = RS // NW
CH = 6400
NCH = GEN_VP // CH
IDS_PER = B * SP // NW


def _tc_body(*refs):
    (xbf_ref, wg_ref, bg_ref, xf_ref, wp_ref, bp_ref, sc_ref) = refs[:7]
    e_refs = refs[7:7 + NSPLIT]
    sg_ref, pv_ref, z_ref = refs[7 + NSPLIT:]
    i = pl.program_id(0)
    s = jnp.dot(xbf_ref[...], wg_ref[...], preferred_element_type=jnp.float32)
    s = s + bg_ref[...]
    e = jnp.exp(s)
    col = i * TN + lax.broadcasted_iota(jnp.int32, e.shape, 1)
    e = jnp.where(col < GEN_V, e, 0.0)
    eb = e.astype(jnp.bfloat16)
    for k, er in enumerate(e_refs):
        er[...] = eb[k * RS:(k + 1) * RS]

    @pl.when(i == 0)
    def _():
        z_ref[...] = jnp.zeros_like(z_ref)

    z_ref[:, 0:1] += jnp.sum(e, axis=1, keepdims=True)

    @pl.when(i == NT - 1)
    def _():
        interp = jax.nn.sigmoid(
            jnp.dot(xf_ref[...], wp_ref[...], preferred_element_type=jnp.float32)
            + bp_ref[0, 0])
        sg_ref[...] = jnp.broadcast_to(interp / z_ref[:, 0:1], (B, L))
        sc = sc_ref[...]
        m = jnp.max(sc, axis=1, keepdims=True)
        es = jnp.exp(sc - m)
        a = es / jnp.sum(es, axis=1, keepdims=True)
        pv_ref[:, 0:S] = (1.0 - interp) * a
        pv_ref[:, S:SP] = jnp.zeros((B, SP - S), jnp.float32)


def _tc_stage(x_bf, wg_bf, bg2, x, wp, bp2, scores):
    return pl.pallas_call(
        _tc_body,
        grid=(NT,),
        in_specs=[
            pl.BlockSpec((B, D), lambda i: (0, 0)),
            pl.BlockSpec((D, TN), lambda i: (0, i)),
            pl.BlockSpec((1, TN), lambda i: (0, i)),
            pl.BlockSpec((B, D), lambda i: (0, 0)),
            pl.BlockSpec((D, 1), lambda i: (0, 0)),
            pl.BlockSpec((1, 1), lambda i: (0, 0)),
            pl.BlockSpec((B, S), lambda i: (0, 0)),
        ],
        out_specs=[pl.BlockSpec((RS, TN), lambda i: (0, i))] * NSPLIT + [
            pl.BlockSpec((B, L), lambda i: (0, 0)),
            pl.BlockSpec((B, SP), lambda i: (0, 0)),
        ],
        out_shape=[jax.ShapeDtypeStruct((RS, GEN_VP), jnp.bfloat16)] * NSPLIT + [
            jax.ShapeDtypeStruct((B, L), jnp.float32),
            jax.ShapeDtypeStruct((B, SP), jnp.float32),
        ],
        scratch_shapes=[pltpu.VMEM((B, 8), jnp.float32)],
    )(x_bf, wg_bf, bg2, x, wp, bp2, scores)


_MESH = plsc.VectorSubcoreMesh(core_axis_name="c", subcore_axis_name="s")
_SC_PARAMS = pltpu.CompilerParams(
    needs_layout_passes=False, use_tc_tiling_on_sc=False)


@functools.partial(
    pl.kernel,
    mesh=_MESH,
    compiler_params=_SC_PARAMS,
    out_type=jax.ShapeDtypeStruct((B * SP,), jnp.int32),
    scratch_types=[
        pltpu.VMEM((INP_V,), jnp.int32),
        pltpu.VMEM((IDS_PER,), jnp.int32),
        pltpu.VMEM((IDS_PER,), jnp.int32),
    ],
)
def _sc_remap(tbl_hbm, ctx_hbm, out_hbm, tbl_v, in_v, out_v):
    wid = lax.axis_index("s") * NC + lax.axis_index("c")
    base = wid * IDS_PER
    pltpu.sync_copy(tbl_hbm, tbl_v)
    pltpu.sync_copy(ctx_hbm.at[pl.ds(base, IDS_PER)], in_v)

    def body(i, carry):
        idx = in_v[pl.ds(i * L, L)]
        out_v[pl.ds(i * L, L)] = plsc.load_gather(tbl_v, [idx])
        return carry

    lax.fori_loop(0, IDS_PER // L, body, 0, unroll=8)
    pltpu.sync_copy(out_v, out_hbm.at[pl.ds(base, IDS_PER)])


def _make_scatter(base):
    @functools.partial(
        pl.kernel,
        mesh=_MESH,
        compiler_params=_SC_PARAMS,
        out_type=jax.ShapeDtypeStruct((RS, OUT_V), jnp.float32),
        scratch_types=[
            pltpu.VMEM((OUT_V,), jnp.float32),
            pltpu.VMEM((CH,), jnp.bfloat16),
            pltpu.VMEM((CH,), jnp.int32),
            pltpu.VMEM((CH,), jnp.bfloat16),
            pltpu.VMEM((CH,), jnp.int32),
            pltpu.VMEM((ROWS_PER, SP), jnp.float32),
            pltpu.VMEM((ROWS_PER * SP,), jnp.int32),
            pltpu.VMEM((ROWS_PER, L), jnp.float32),
            pltpu.VMEM_SHARED((GEN_VP,), jnp.int32),
            pltpu.SemaphoreType.DMA,
            pltpu.SemaphoreType.DMA,
            pltpu.SemaphoreType.DMA,
            pltpu.SemaphoreType.DMA,
            pltpu.SemaphoreType.DMA,
        ],
    )
    def _sc_scatter(e_hbm, g2o_hbm, ctxo_hbm, pv_hbm, sg_hbm, out_hbm,
                    row_v, val_a, idx_a, val_b, idx_b, pv_v, pi_v, sg_v,
                    g2o_s, sem_va, sem_ia, sem_vb, sem_ib, sem_out):
        cid = lax.axis_index("c")
        sid = lax.axis_index("s")
        wid = sid * NC + cid
        lanes = lax.iota(jnp.int32, L)
        last_mask = lanes < (S - (SP // L - 1) * L)

        pltpu.sync_copy(g2o_hbm.at[pl.ds(sid * (GEN_VP // NS), GEN_VP // NS)],
                        g2o_s.at[pl.ds(sid * (GEN_VP // NS), GEN_VP // NS)])

        gbase = base + wid * ROWS_PER
        pltpu.sync_copy(sg_hbm.at[pl.ds(gbase, ROWS_PER)], sg_v)
        pltpu.sync_copy(pv_hbm.at[pl.ds(gbase, ROWS_PER)], pv_v)
        pltpu.sync_copy(ctxo_hbm.at[pl.ds(gbase * SP, ROWS_PER * SP)], pi_v)
        plsc.subcore_barrier()

        def start(b, c, val_v, idx_v, sem_v, sem_i):
            pltpu.async_copy(e_hbm.at[b, pl.ds(c * CH, CH)], val_v, sem_v)
            pltpu.async_copy(g2o_s.at[pl.ds(c * CH, CH)], idx_v, sem_i)

        def wait(b, val_v, idx_v, sem_v, sem_i):
            pltpu.make_async_copy(
                e_hbm.at[b, pl.ds(0, CH)], val_v, sem_v).wait()
            pltpu.make_async_copy(
                g2o_s.at[pl.ds(0, CH)], idx_v, sem_i).wait()

        def scatter_chunk(val_v, idx_v, sg):
            @plsc.parallel_loop(0, CH // (2 * L), unroll=8)
            def vbody(i):
                ev = val_v[pl.ds(i * 2 * L, 2 * L)]
                va, vb = plsc.unpack(ev, format=plsc.PackFormat.INTERLEAVED)
                ia = idx_v[pl.ds(i * 2 * L, L)]
                ib = idx_v[pl.ds(i * 2 * L + L, L)]
                plsc.addupdate_scatter(row_v, [ia], va * sg)
                plsc.addupdate_scatter(row_v, [ib], vb * sg)

        def row_body(r, carry):
            bl = wid * ROWS_PER + r
            b = base + bl

            start(bl, 0, val_a, idx_a, sem_va, sem_ia)

            pltpu.make_async_copy(row_v, out_hbm.at[bl], sem_out).wait()

            @plsc.parallel_loop(0, OUT_V // L, unroll=16)
            def zbody(i):
                row_v[pl.ds(i * L, L)] = jnp.zeros((L,), jnp.float32)

            sg = sg_v[r, :]

            def pair_body(g, carry2):
                start(bl, 2 * g + 1, val_b, idx_b, sem_vb, sem_ib)
                wait(bl, val_a, idx_a, sem_va, sem_ia)
                scatter_chunk(val_a, idx_a, sg)

                @pl.when(g < NCH // 2 - 1)
                def _():
                    start(bl, 2 * g + 2, val_a, idx_a, sem_va, sem_ia)

                wait(bl, val_b, idx_b, sem_vb, sem_ib)
                scatter_chunk(val_b, idx_b, sg)
                return carry2

            lax.fori_loop(0, NCH // 2, pair_body, 0)

            for j in range(SP // L):
                iv = pi_v[pl.ds(r * SP + j * L, L)]
                vv = pv_v[r, pl.ds(j * L, L)]
                if j < SP // L - 1:
                    plsc.addupdate_scatter(row_v, [iv], vv)
                else:
                    plsc.addupdate_scatter(row_v, [iv], vv, mask=last_mask)

            pltpu.async_copy(row_v, out_hbm.at[bl], sem_out)
            return carry

        pltpu.async_copy(row_v, out_hbm.at[wid * ROWS_PER], sem_out)
        lax.fori_loop(0, ROWS_PER, row_body, 0)
        pltpu.make_async_copy(row_v, out_hbm.at[0], sem_out).wait()

    return _sc_scatter


_SCATTERS = [_make_scatter(k * RS) for k in range(NSPLIT)]


def kernel(x, scores, ctx_inp, Wp, bp, Wg, bg, gen_to_out, inp_to_out):
    x_bf = x.astype(jnp.bfloat16)
    wg_bf = Wg.astype(jnp.bfloat16)
    bg2 = bg.reshape(1, GEN_V)
    bp2 = bp.reshape(1, 1)
    ctx_pad = jnp.pad(ctx_inp, ((0, 0), (0, SP - S))).reshape(-1)
    g2o_p = jnp.pad(gen_to_out, (0, GEN_VP - GEN_V))
    g2o_r = g2o_p.reshape(-1, L, 2).transpose(0, 2, 1).reshape(-1)

    *e_pieces, sg, pv = _tc_stage(x_bf, wg_bf, bg2, x, Wp, bp2, scores)
    ctx_out = _sc_remap(inp_to_out, ctx_pad)
    outs = [s(ek, g2o_r, ctx_out, pv, sg)
            for s, ek in zip(_SCATTERS, e_pieces)]
    return jnp.concatenate(outs, axis=0)
